# Initial kernel scaffold; baseline (speedup 1.0000x reference)
import jax
import jax.numpy as jnp
from jax.experimental import pallas as pl


def kernel(x, edge_index, W1, b1, W2, b2):
    raise NotImplementedError("write your pallas kernel here")


# speedup vs baseline: 92.5213x; 92.5213x over previous
import functools

import jax
import jax.numpy as jnp
from jax import lax
from jax.experimental import pallas as pl
from jax.experimental.pallas import tpu as pltpu
from jax.experimental.pallas import tpu_sc as plsc

NC = 2
NS = 16
NW = NC * NS
CHUNK = 2000
DP = 8

_SC_PARAMS = dict(
    mesh=plsc.VectorSubcoreMesh(
        core_axis_name="c", subcore_axis_name="s",
        num_cores=NC, num_subcores=NS),
    compiler_params=pltpu.CompilerParams(use_tc_tiling_on_sc=False),
)


def _make_degree(n, e):
    epw = e // NW
    iters = epw // CHUNK

    def body(dst_hbm, zeros_hbm, ones_hbm, out_hbm, idx_v, ones_v, acc_sh, sem):
        c = lax.axis_index("c")
        s = lax.axis_index("s")
        wid = c * NS + s
        pltpu.sync_copy(ones_hbm, ones_v)

        @pl.when(s == 0)
        def _():
            pltpu.sync_copy(zeros_hbm, acc_sh)

        plsc.subcore_barrier()

        def it(i, carry):
            base = wid * epw + i * CHUNK
            pltpu.sync_copy(dst_hbm.at[pl.ds(base, CHUNK)], idx_v)
            pltpu.sync_copy(ones_v, acc_sh.at[idx_v], add=True)
            return carry

        lax.fori_loop(0, iters, it, 0)
        plsc.subcore_barrier()

        @pl.when(s == 0)
        def _():
            pltpu.sync_copy(acc_sh, out_hbm.at[c])

    return pl.kernel(
        body,
        out_type=jax.ShapeDtypeStruct((NC, n, DP), jnp.float32),
        scratch_types=[
            pltpu.VMEM((CHUNK,), jnp.int32),
            pltpu.VMEM((CHUNK, DP), jnp.float32),
            pltpu.VMEM_SHARED((n, DP), jnp.float32),
            pltpu.SemaphoreType.DMA,
        ],
        **_SC_PARAMS,
    )


def _make_agg(n, e):
    epw = e // NW
    iters = epw // CHUNK

    def body(src_hbm, dst_hbm, xs_hbm, zeros_hbm, out_hbm,
             idxs_v, idxd_v, rows_v, acc_sh, sem):
        c = lax.axis_index("c")
        s = lax.axis_index("s")
        wid = c * NS + s

        @pl.when(s == 0)
        def _():
            pltpu.sync_copy(zeros_hbm, acc_sh)

        plsc.subcore_barrier()

        def it(i, carry):
            base = wid * epw + i * CHUNK
            pltpu.sync_copy(src_hbm.at[pl.ds(base, CHUNK)], idxs_v)
            pltpu.async_copy(xs_hbm.at[idxs_v], rows_v, sem).wait()
            pltpu.sync_copy(dst_hbm.at[pl.ds(base, CHUNK)], idxd_v)
            pltpu.sync_copy(rows_v, acc_sh.at[idxd_v], add=True)
            return carry

        lax.fori_loop(0, iters, it, 0)
        plsc.subcore_barrier()

        @pl.when(s == 0)
        def _():
            pltpu.sync_copy(acc_sh, out_hbm.at[c])

    return pl.kernel(
        body,
        out_type=jax.ShapeDtypeStruct((NC, n, DP), jnp.float32),
        scratch_types=[
            pltpu.VMEM((CHUNK,), jnp.int32),
            pltpu.VMEM((CHUNK,), jnp.int32),
            pltpu.VMEM((CHUNK, DP), jnp.float32),
            pltpu.VMEM_SHARED((n, DP), jnp.float32),
            pltpu.SemaphoreType.DMA,
        ],
        **_SC_PARAMS,
    )


_BN = 5000


def _tc1_body(deg2_ref, x_ref, dinv_ref, xs_ref):
    deg = deg2_ref[0, :, :4] + deg2_ref[1, :, :4] + 1.0
    dinv = lax.rsqrt(deg)
    dinv_ref[...] = dinv
    xs_ref[:, :4] = dinv * x_ref[...]
    xs_ref[:, 4:] = jnp.zeros_like(dinv)


def _mm4(a, w_ref, b_ref):
    z = b_ref[...]
    for k in range(4):
        z = z + a[:, k:k + 1] * w_ref[k:k + 1, :]
    return z


def _tc2_body(s1_ref, x_ref, dinv_ref, w1_ref, b1_ref, h1_ref, xs2_ref):
    dinv = dinv_ref[...]
    ssum = s1_ref[0, :, :4] + s1_ref[1, :, :4]
    agg = dinv * ssum + dinv * dinv * x_ref[...]
    h1 = jnp.maximum(_mm4(agg, w1_ref, b1_ref), 0.0)
    h1_ref[...] = h1
    xs2_ref[:, :4] = dinv * h1
    xs2_ref[:, 4:] = jnp.zeros_like(dinv)


def _tc3_body(s2_ref, h1_ref, dinv_ref, w2_ref, b2_ref, out_ref):
    dinv = dinv_ref[...]
    ssum = s2_ref[0, :, :4] + s2_ref[1, :, :4]
    agg = dinv * ssum + dinv * dinv * h1_ref[...]
    z = _mm4(agg, w2_ref, b2_ref)
    m = jnp.max(z, axis=1, keepdims=True)
    ez = jnp.exp(z - m)
    sz = jnp.sum(ez, axis=1, keepdims=True)
    out_ref[...] = z - m - jnp.log(sz)


def _nd_spec(w):
    return pl.BlockSpec((_BN, w), lambda i: (i, 0))


def _p2_spec():
    return pl.BlockSpec((NC, _BN, DP), lambda i: (0, i, 0))


def _w_spec():
    return pl.BlockSpec((4, 4), lambda i: (0, 0))


def _b_spec():
    return pl.BlockSpec((1, 4), lambda i: (0, 0))


def kernel(x, edge_index, W1, b1, W2, b2):
    n, d = x.shape
    e = edge_index.shape[1]
    src = edge_index[0]
    dst = edge_index[1]
    zeros8 = jnp.zeros((n, DP), jnp.float32)
    ones8 = jnp.ones((CHUNK, DP), jnp.float32)

    deg2 = _make_degree(n, e)(dst, zeros8, ones8)

    grid = (n // _BN,)
    f32 = jnp.float32

    dinv, xs1 = pl.pallas_call(
        _tc1_body,
        grid=grid,
        in_specs=[_p2_spec(), _nd_spec(4)],
        out_specs=[_nd_spec(4), _nd_spec(DP)],
        out_shape=[jax.ShapeDtypeStruct((n, 4), f32),
                   jax.ShapeDtypeStruct((n, DP), f32)],
    )(deg2, x)

    s1 = _make_agg(n, e)(src, dst, xs1, zeros8)

    h1, xs2 = pl.pallas_call(
        _tc2_body,
        grid=grid,
        in_specs=[_p2_spec(), _nd_spec(4), _nd_spec(4), _w_spec(), _b_spec()],
        out_specs=[_nd_spec(4), _nd_spec(DP)],
        out_shape=[jax.ShapeDtypeStruct((n, 4), f32),
                   jax.ShapeDtypeStruct((n, DP), f32)],
    )(s1, x, dinv, W1, b1.reshape(1, 4))

    s2 = _make_agg(n, e)(src, dst, xs2, zeros8)

    out = pl.pallas_call(
        _tc3_body,
        grid=grid,
        in_specs=[_p2_spec(), _nd_spec(4), _nd_spec(4), _w_spec(), _b_spec()],
        out_specs=_nd_spec(4),
        out_shape=jax.ShapeDtypeStruct((n, 4), f32),
    )(s2, h1, dinv, W2, b2.reshape(1, 4))
    return out


# speedup vs baseline: 110.4230x; 1.1935x over previous
import functools

import jax
import jax.numpy as jnp
from jax import lax
from jax.experimental import pallas as pl
from jax.experimental.pallas import tpu as pltpu
from jax.experimental.pallas import tpu_sc as plsc

NC = 2
NS = 16
NW = NC * NS
CHUNK = 8000
DP = 8

_SC_PARAMS = dict(
    mesh=plsc.VectorSubcoreMesh(
        core_axis_name="c", subcore_axis_name="s",
        num_cores=NC, num_subcores=NS),
    compiler_params=pltpu.CompilerParams(use_tc_tiling_on_sc=False),
)


def _make_degree(n, e):
    epw = e // NW
    iters = epw // CHUNK

    def body(dst_hbm, zeros_hbm, ones_hbm, out_hbm, idx_v, ones_v, acc_sh, sem):
        c = lax.axis_index("c")
        s = lax.axis_index("s")
        wid = c * NS + s
        pltpu.sync_copy(ones_hbm, ones_v)

        @pl.when(s == 0)
        def _():
            pltpu.sync_copy(zeros_hbm, acc_sh)

        plsc.subcore_barrier()

        def it(i, carry):
            base = wid * epw + i * CHUNK
            pltpu.sync_copy(dst_hbm.at[pl.ds(base, CHUNK)], idx_v)
            pltpu.sync_copy(ones_v, acc_sh.at[idx_v], add=True)
            return carry

        lax.fori_loop(0, iters, it, 0)
        plsc.subcore_barrier()

        @pl.when(s == 0)
        def _():
            pltpu.sync_copy(acc_sh, out_hbm.at[c])

    return pl.kernel(
        body,
        out_type=jax.ShapeDtypeStruct((NC, n, DP), jnp.float32),
        scratch_types=[
            pltpu.VMEM((CHUNK,), jnp.int32),
            pltpu.VMEM((CHUNK, DP), jnp.float32),
            pltpu.VMEM_SHARED((n, DP), jnp.float32),
            pltpu.SemaphoreType.DMA,
        ],
        **_SC_PARAMS,
    )


def _make_agg(n, e):
    epw = e // NW
    iters = epw // CHUNK

    def body(src_hbm, dst_hbm, xs_hbm, zeros_hbm, out_hbm,
             idxs_v, idxd_v, rows_v, acc_sh, sem):
        c = lax.axis_index("c")
        s = lax.axis_index("s")
        wid = c * NS + s

        @pl.when(s == 0)
        def _():
            pltpu.sync_copy(zeros_hbm, acc_sh)

        plsc.subcore_barrier()

        def it(i, carry):
            base = wid * epw + i * CHUNK
            pltpu.sync_copy(src_hbm.at[pl.ds(base, CHUNK)], idxs_v)
            pltpu.async_copy(xs_hbm.at[idxs_v], rows_v, sem).wait()
            pltpu.sync_copy(dst_hbm.at[pl.ds(base, CHUNK)], idxd_v)
            pltpu.sync_copy(rows_v, acc_sh.at[idxd_v], add=True)
            return carry

        lax.fori_loop(0, iters, it, 0)
        plsc.subcore_barrier()

        @pl.when(s == 0)
        def _():
            pltpu.sync_copy(acc_sh, out_hbm.at[c])

    return pl.kernel(
        body,
        out_type=jax.ShapeDtypeStruct((NC, n, DP), jnp.float32),
        scratch_types=[
            pltpu.VMEM((CHUNK,), jnp.int32),
            pltpu.VMEM((CHUNK,), jnp.int32),
            pltpu.VMEM((CHUNK, DP), jnp.float32),
            pltpu.VMEM_SHARED((n, DP), jnp.float32),
            pltpu.SemaphoreType.DMA,
        ],
        **_SC_PARAMS,
    )


_BN = 5000


def _tc1_body(deg2_ref, x_ref, dinv_ref, xs_ref):
    deg = deg2_ref[0, :, :4] + deg2_ref[1, :, :4] + 1.0
    dinv = lax.rsqrt(deg)
    dinv_ref[...] = dinv
    xs_ref[:, :4] = dinv * x_ref[...]
    xs_ref[:, 4:] = jnp.zeros_like(dinv)


def _mm4(a, w_ref, b_ref):
    z = b_ref[...]
    for k in range(4):
        z = z + a[:, k:k + 1] * w_ref[k:k + 1, :]
    return z


def _tc2_body(s1_ref, x_ref, dinv_ref, w1_ref, b1_ref, h1_ref, xs2_ref):
    dinv = dinv_ref[...]
    ssum = s1_ref[0, :, :4] + s1_ref[1, :, :4]
    agg = dinv * ssum + dinv * dinv * x_ref[...]
    h1 = jnp.maximum(_mm4(agg, w1_ref, b1_ref), 0.0)
    h1_ref[...] = h1
    xs2_ref[:, :4] = dinv * h1
    xs2_ref[:, 4:] = jnp.zeros_like(dinv)


def _tc3_body(s2_ref, h1_ref, dinv_ref, w2_ref, b2_ref, out_ref):
    dinv = dinv_ref[...]
    ssum = s2_ref[0, :, :4] + s2_ref[1, :, :4]
    agg = dinv * ssum + dinv * dinv * h1_ref[...]
    z = _mm4(agg, w2_ref, b2_ref)
    m = jnp.max(z, axis=1, keepdims=True)
    ez = jnp.exp(z - m)
    sz = jnp.sum(ez, axis=1, keepdims=True)
    out_ref[...] = z - m - jnp.log(sz)


def _nd_spec(w):
    return pl.BlockSpec((_BN, w), lambda i: (i, 0))


def _p2_spec():
    return pl.BlockSpec((NC, _BN, DP), lambda i: (0, i, 0))


def _w_spec():
    return pl.BlockSpec((4, 4), lambda i: (0, 0))


def _b_spec():
    return pl.BlockSpec((1, 4), lambda i: (0, 0))


def kernel(x, edge_index, W1, b1, W2, b2):
    n, d = x.shape
    e = edge_index.shape[1]
    src = edge_index[0]
    dst = edge_index[1]
    zeros8 = jnp.zeros((n, DP), jnp.float32)
    ones8 = jnp.ones((CHUNK, DP), jnp.float32)

    deg2 = _make_degree(n, e)(dst, zeros8, ones8)

    grid = (n // _BN,)
    f32 = jnp.float32

    dinv, xs1 = pl.pallas_call(
        _tc1_body,
        grid=grid,
        in_specs=[_p2_spec(), _nd_spec(4)],
        out_specs=[_nd_spec(4), _nd_spec(DP)],
        out_shape=[jax.ShapeDtypeStruct((n, 4), f32),
                   jax.ShapeDtypeStruct((n, DP), f32)],
    )(deg2, x)

    s1 = _make_agg(n, e)(src, dst, xs1, zeros8)

    h1, xs2 = pl.pallas_call(
        _tc2_body,
        grid=grid,
        in_specs=[_p2_spec(), _nd_spec(4), _nd_spec(4), _w_spec(), _b_spec()],
        out_specs=[_nd_spec(4), _nd_spec(DP)],
        out_shape=[jax.ShapeDtypeStruct((n, 4), f32),
                   jax.ShapeDtypeStruct((n, DP), f32)],
    )(s1, x, dinv, W1, b1.reshape(1, 4))

    s2 = _make_agg(n, e)(src, dst, xs2, zeros8)

    out = pl.pallas_call(
        _tc3_body,
        grid=grid,
        in_specs=[_p2_spec(), _nd_spec(4), _nd_spec(4), _w_spec(), _b_spec()],
        out_specs=_nd_spec(4),
        out_shape=jax.ShapeDtypeStruct((n, 4), f32),
    )(s2, h1, dinv, W2, b2.reshape(1, 4))
    return out


# speedup vs baseline: 125.4966x; 1.1365x over previous
import functools

import jax
import jax.numpy as jnp
from jax import lax
from jax.experimental import pallas as pl
from jax.experimental.pallas import tpu as pltpu
from jax.experimental.pallas import tpu_sc as plsc

NC = 2
NS = 16
NW = NC * NS
CHUNK = 5000
CHUNK_A = 4000
DP = 8

_SC_PARAMS = dict(
    mesh=plsc.VectorSubcoreMesh(
        core_axis_name="c", subcore_axis_name="s",
        num_cores=NC, num_subcores=NS),
    compiler_params=pltpu.CompilerParams(use_tc_tiling_on_sc=False),
)


def _make_degree(n, e):
    epw = e // NW
    iters = epw // CHUNK

    def body(dst_hbm, zeros_hbm, ones_hbm, out_hbm,
             idx0, idx1, ones_v, acc_sh, isem0, isem1, ssem0, ssem1):
        c = lax.axis_index("c")
        s = lax.axis_index("s")
        wid = c * NS + s
        e0 = wid * epw
        pltpu.sync_copy(ones_hbm, ones_v)

        @pl.when(s == 0)
        def _():
            pltpu.sync_copy(zeros_hbm, acc_sh)

        plsc.subcore_barrier()
        bufs = ((idx0, isem0, ssem0), (idx1, isem1, ssem1))

        def issue_idx(i, idx, isem):
            pltpu.async_copy(dst_hbm.at[pl.ds(e0 + i * CHUNK, CHUNK)], idx, isem)

        def wait_idx(idx, isem):
            pltpu.make_async_copy(dst_hbm.at[pl.ds(0, CHUNK)], idx, isem).wait()

        issue_idx(0, idx0, isem0)

        def pair(i2, carry):
            for b in (0, 1):
                idx, isem, ssem = bufs[b]
                oidx, oisem, ossem = bufs[1 - b]
                i = i2 * 2 + b
                wait_idx(idx, isem)
                pltpu.async_copy(ones_v, acc_sh.at[idx], ssem, add=True)
                if b == 0:
                    @pl.when(i2 >= 1)
                    def _():
                        pltpu.make_async_copy(ones_v, acc_sh.at[oidx], ossem).wait()
                    issue_idx(i + 1, oidx, oisem)
                else:
                    pltpu.make_async_copy(ones_v, acc_sh.at[oidx], ossem).wait()

                    @pl.when(i2 < iters // 2 - 1)
                    def _():
                        issue_idx(i + 1, oidx, oisem)
            return carry

        lax.fori_loop(0, iters // 2, pair, 0)
        pltpu.make_async_copy(ones_v, acc_sh.at[idx1], ssem1).wait()
        plsc.subcore_barrier()

        @pl.when(s == 0)
        def _():
            pltpu.sync_copy(acc_sh, out_hbm.at[c])

    return pl.kernel(
        body,
        out_type=jax.ShapeDtypeStruct((NC, n, DP), jnp.float32),
        scratch_types=[
            pltpu.VMEM((CHUNK,), jnp.int32),
            pltpu.VMEM((CHUNK,), jnp.int32),
            pltpu.VMEM((CHUNK, DP), jnp.float32),
            pltpu.VMEM_SHARED((n, DP), jnp.float32),
            pltpu.SemaphoreType.DMA,
            pltpu.SemaphoreType.DMA,
            pltpu.SemaphoreType.DMA,
            pltpu.SemaphoreType.DMA,
        ],
        **_SC_PARAMS,
    )


def _make_agg(n, e):
    epw = e // NW
    iters = epw // CHUNK_A

    def body(src_hbm, dst_hbm, xs_hbm, zeros_hbm, out_hbm,
             idxs0, idxs1, idxd0, idxd1, rows0, rows1, acc_sh,
             isem0, isem1, gsem0, gsem1, ssem0, ssem1):
        c = lax.axis_index("c")
        s = lax.axis_index("s")
        wid = c * NS + s
        e0 = wid * epw

        @pl.when(s == 0)
        def _():
            pltpu.sync_copy(zeros_hbm, acc_sh)

        plsc.subcore_barrier()
        bufs = ((idxs0, idxd0, rows0, isem0, gsem0, ssem0),
                (idxs1, idxd1, rows1, isem1, gsem1, ssem1))

        def issue_idx(i, idxs, idxd, isem):
            base = e0 + i * CHUNK_A
            pltpu.async_copy(src_hbm.at[pl.ds(base, CHUNK_A)], idxs, isem)
            pltpu.async_copy(dst_hbm.at[pl.ds(base, CHUNK_A)], idxd, isem)

        def wait_idx(idxs, idxd, isem):
            pltpu.make_async_copy(src_hbm.at[pl.ds(0, CHUNK_A)], idxs, isem).wait()
            pltpu.make_async_copy(dst_hbm.at[pl.ds(0, CHUNK_A)], idxd, isem).wait()

        issue_idx(0, idxs0, idxd0, isem0)

        def pair(i2, carry):
            for b in (0, 1):
                idxs, idxd, rows, isem, gsem, ssem = bufs[b]
                oidxs, oidxd, orows, oisem, ogsem, ossem = bufs[1 - b]
                wait_idx(idxs, idxd, isem)
                g = pltpu.async_copy(xs_hbm.at[idxs], rows, gsem)
                if b == 0:
                    @pl.when(i2 >= 1)
                    def _():
                        pltpu.make_async_copy(orows, acc_sh.at[oidxd], ossem).wait()
                    issue_idx(i2 * 2 + 1, oidxs, oidxd, oisem)
                else:
                    pltpu.make_async_copy(orows, acc_sh.at[oidxd], ossem).wait()

                    @pl.when(i2 < iters // 2 - 1)
                    def _():
                        issue_idx(i2 * 2 + 2, oidxs, oidxd, oisem)
                g.wait()
                pltpu.async_copy(rows, acc_sh.at[idxd], ssem, add=True)
            return carry

        lax.fori_loop(0, iters // 2, pair, 0)
        pltpu.make_async_copy(rows1, acc_sh.at[idxd1], ssem1).wait()
        plsc.subcore_barrier()

        @pl.when(s == 0)
        def _():
            pltpu.sync_copy(acc_sh, out_hbm.at[c])

    return pl.kernel(
        body,
        out_type=jax.ShapeDtypeStruct((NC, n, DP), jnp.float32),
        scratch_types=[
            pltpu.VMEM((CHUNK_A,), jnp.int32),
            pltpu.VMEM((CHUNK_A,), jnp.int32),
            pltpu.VMEM((CHUNK_A,), jnp.int32),
            pltpu.VMEM((CHUNK_A,), jnp.int32),
            pltpu.VMEM((CHUNK_A, DP), jnp.float32),
            pltpu.VMEM((CHUNK_A, DP), jnp.float32),
            pltpu.VMEM_SHARED((n, DP), jnp.float32),
            pltpu.SemaphoreType.DMA,
            pltpu.SemaphoreType.DMA,
            pltpu.SemaphoreType.DMA,
            pltpu.SemaphoreType.DMA,
            pltpu.SemaphoreType.DMA,
            pltpu.SemaphoreType.DMA,
        ],
        **_SC_PARAMS,
    )


_BN = 5000


def _tc1_body(deg2_ref, x_ref, dinv_ref, xs_ref):
    deg = deg2_ref[0, :, :4] + deg2_ref[1, :, :4] + 1.0
    dinv = lax.rsqrt(deg)
    dinv_ref[...] = dinv
    xs_ref[:, :4] = dinv * x_ref[...]
    xs_ref[:, 4:] = jnp.zeros_like(dinv)


def _mm4(a, w_ref, b_ref):
    z = b_ref[...]
    for k in range(4):
        z = z + a[:, k:k + 1] * w_ref[k:k + 1, :]
    return z


def _tc2_body(s1_ref, x_ref, dinv_ref, w1_ref, b1_ref, h1_ref, xs2_ref):
    dinv = dinv_ref[...]
    ssum = s1_ref[0, :, :4] + s1_ref[1, :, :4]
    agg = dinv * ssum + dinv * dinv * x_ref[...]
    h1 = jnp.maximum(_mm4(agg, w1_ref, b1_ref), 0.0)
    h1_ref[...] = h1
    xs2_ref[:, :4] = dinv * h1
    xs2_ref[:, 4:] = jnp.zeros_like(dinv)


def _tc3_body(s2_ref, h1_ref, dinv_ref, w2_ref, b2_ref, out_ref):
    dinv = dinv_ref[...]
    ssum = s2_ref[0, :, :4] + s2_ref[1, :, :4]
    agg = dinv * ssum + dinv * dinv * h1_ref[...]
    z = _mm4(agg, w2_ref, b2_ref)
    m = jnp.max(z, axis=1, keepdims=True)
    ez = jnp.exp(z - m)
    sz = jnp.sum(ez, axis=1, keepdims=True)
    out_ref[...] = z - m - jnp.log(sz)


def _nd_spec(w):
    return pl.BlockSpec((_BN, w), lambda i: (i, 0))


def _p2_spec():
    return pl.BlockSpec((NC, _BN, DP), lambda i: (0, i, 0))


def _w_spec():
    return pl.BlockSpec((4, 4), lambda i: (0, 0))


def _b_spec():
    return pl.BlockSpec((1, 4), lambda i: (0, 0))


def kernel(x, edge_index, W1, b1, W2, b2):
    n, d = x.shape
    e = edge_index.shape[1]
    src = edge_index[0]
    dst = edge_index[1]
    zeros8 = jnp.zeros((n, DP), jnp.float32)
    ones8 = jnp.ones((CHUNK, DP), jnp.float32)

    deg2 = _make_degree(n, e)(dst, zeros8, ones8)

    grid = (n // _BN,)
    f32 = jnp.float32

    dinv, xs1 = pl.pallas_call(
        _tc1_body,
        grid=grid,
        in_specs=[_p2_spec(), _nd_spec(4)],
        out_specs=[_nd_spec(4), _nd_spec(DP)],
        out_shape=[jax.ShapeDtypeStruct((n, 4), f32),
                   jax.ShapeDtypeStruct((n, DP), f32)],
    )(deg2, x)

    s1 = _make_agg(n, e)(src, dst, xs1, zeros8)

    h1, xs2 = pl.pallas_call(
        _tc2_body,
        grid=grid,
        in_specs=[_p2_spec(), _nd_spec(4), _nd_spec(4), _w_spec(), _b_spec()],
        out_specs=[_nd_spec(4), _nd_spec(DP)],
        out_shape=[jax.ShapeDtypeStruct((n, 4), f32),
                   jax.ShapeDtypeStruct((n, DP), f32)],
    )(s1, x, dinv, W1, b1.reshape(1, 4))

    s2 = _make_agg(n, e)(src, dst, xs2, zeros8)

    out = pl.pallas_call(
        _tc3_body,
        grid=grid,
        in_specs=[_p2_spec(), _nd_spec(4), _nd_spec(4), _w_spec(), _b_spec()],
        out_specs=_nd_spec(4),
        out_shape=jax.ShapeDtypeStruct((n, 4), f32),
    )(s2, h1, dinv, W2, b2.reshape(1, 4))
    return out


# speedup vs baseline: 196.5454x; 1.5661x over previous
import jax
import jax.numpy as jnp
from jax import lax
from jax.experimental import pallas as pl
from jax.experimental.pallas import tpu as pltpu
from jax.experimental.pallas import tpu_sc as plsc

NC = 2
NS = 16
NW = NC * NS
CHUNK = 5000
CHUNK_A = 4000
DP = 8

_SC_PARAMS = dict(
    mesh=plsc.VectorSubcoreMesh(
        core_axis_name="c", subcore_axis_name="s",
        num_cores=NC, num_subcores=NS),
    compiler_params=pltpu.CompilerParams(use_tc_tiling_on_sc=False),
)


def _make_degree(n, e):
    epw = e // NW
    iters = epw // CHUNK

    def body(ei_hbm, zeros_hbm, ones_hbm, out_hbm,
             idx0, idx1, ones_v, acc_sh, isem0, isem1, ssem0, ssem1):
        c = lax.axis_index("c")
        s = lax.axis_index("s")
        wid = c * NS + s
        e0 = wid * epw
        pltpu.sync_copy(ones_hbm, ones_v)

        @pl.when(s == 0)
        def _():
            pltpu.sync_copy(zeros_hbm, acc_sh)

        plsc.subcore_barrier()
        bufs = ((idx0, isem0, ssem0), (idx1, isem1, ssem1))

        def issue_idx(i, idx, isem):
            pltpu.async_copy(ei_hbm.at[1, pl.ds(e0 + i * CHUNK, CHUNK)], idx, isem)

        def wait_idx(idx, isem):
            pltpu.make_async_copy(ei_hbm.at[1, pl.ds(0, CHUNK)], idx, isem).wait()

        issue_idx(0, idx0, isem0)

        def pair(i2, carry):
            for b in (0, 1):
                idx, isem, ssem = bufs[b]
                oidx, oisem, ossem = bufs[1 - b]
                i = i2 * 2 + b
                wait_idx(idx, isem)
                pltpu.async_copy(ones_v, acc_sh.at[idx], ssem, add=True)
                if b == 0:
                    @pl.when(i2 >= 1)
                    def _():
                        pltpu.make_async_copy(ones_v, acc_sh.at[oidx], ossem).wait()
                    issue_idx(i + 1, oidx, oisem)
                else:
                    pltpu.make_async_copy(ones_v, acc_sh.at[oidx], ossem).wait()

                    @pl.when(i2 < iters // 2 - 1)
                    def _():
                        issue_idx(i + 1, oidx, oisem)
            return carry

        lax.fori_loop(0, iters // 2, pair, 0)
        pltpu.make_async_copy(ones_v, acc_sh.at[idx1], ssem1).wait()
        plsc.subcore_barrier()

        @pl.when(s == 0)
        def _():
            pltpu.sync_copy(acc_sh, out_hbm.at[c])

    return pl.kernel(
        body,
        out_type=jax.ShapeDtypeStruct((NC, n, DP), jnp.float32),
        scratch_types=[
            pltpu.VMEM((CHUNK,), jnp.int32),
            pltpu.VMEM((CHUNK,), jnp.int32),
            pltpu.VMEM((CHUNK, DP), jnp.float32),
            pltpu.VMEM_SHARED((n, DP), jnp.float32),
            pltpu.SemaphoreType.DMA,
            pltpu.SemaphoreType.DMA,
            pltpu.SemaphoreType.DMA,
            pltpu.SemaphoreType.DMA,
        ],
        **_SC_PARAMS,
    )


def _make_agg(n, e):
    epw = e // NW
    iters = epw // CHUNK_A

    def body(ei_hbm, xs_hbm, zeros_hbm, out_hbm,
             idxs0, idxs1, idxd0, idxd1, rows0, rows1, acc_sh,
             isem0, isem1, gsem0, gsem1, ssem0, ssem1):
        c = lax.axis_index("c")
        s = lax.axis_index("s")
        wid = c * NS + s
        e0 = wid * epw

        @pl.when(s == 0)
        def _():
            pltpu.sync_copy(zeros_hbm, acc_sh)

        plsc.subcore_barrier()
        bufs = ((idxs0, idxd0, rows0, isem0, gsem0, ssem0),
                (idxs1, idxd1, rows1, isem1, gsem1, ssem1))

        def issue_idx(i, idxs, idxd, isem):
            base = e0 + i * CHUNK_A
            pltpu.async_copy(ei_hbm.at[0, pl.ds(base, CHUNK_A)], idxs, isem)
            pltpu.async_copy(ei_hbm.at[1, pl.ds(base, CHUNK_A)], idxd, isem)

        def wait_idx(idxs, idxd, isem):
            pltpu.make_async_copy(ei_hbm.at[0, pl.ds(0, CHUNK_A)], idxs, isem).wait()
            pltpu.make_async_copy(ei_hbm.at[1, pl.ds(0, CHUNK_A)], idxd, isem).wait()

        issue_idx(0, idxs0, idxd0, isem0)

        def pair(i2, carry):
            for b in (0, 1):
                idxs, idxd, rows, isem, gsem, ssem = bufs[b]
                oidxs, oidxd, orows, oisem, ogsem, ossem = bufs[1 - b]
                wait_idx(idxs, idxd, isem)
                g = pltpu.async_copy(xs_hbm.at[idxs], rows, gsem)
                if b == 0:
                    @pl.when(i2 >= 1)
                    def _():
                        pltpu.make_async_copy(orows, acc_sh.at[oidxd], ossem).wait()
                    issue_idx(i2 * 2 + 1, oidxs, oidxd, oisem)
                else:
                    pltpu.make_async_copy(orows, acc_sh.at[oidxd], ossem).wait()

                    @pl.when(i2 < iters // 2 - 1)
                    def _():
                        issue_idx(i2 * 2 + 2, oidxs, oidxd, oisem)
                g.wait()
                pltpu.async_copy(rows, acc_sh.at[idxd], ssem, add=True)
            return carry

        lax.fori_loop(0, iters // 2, pair, 0)
        pltpu.make_async_copy(rows1, acc_sh.at[idxd1], ssem1).wait()
        plsc.subcore_barrier()

        @pl.when(s == 0)
        def _():
            pltpu.sync_copy(acc_sh, out_hbm.at[c])

    return pl.kernel(
        body,
        out_type=jax.ShapeDtypeStruct((NC, n, DP), jnp.float32),
        scratch_types=[
            pltpu.VMEM((CHUNK_A,), jnp.int32),
            pltpu.VMEM((CHUNK_A,), jnp.int32),
            pltpu.VMEM((CHUNK_A,), jnp.int32),
            pltpu.VMEM((CHUNK_A,), jnp.int32),
            pltpu.VMEM((CHUNK_A, DP), jnp.float32),
            pltpu.VMEM((CHUNK_A, DP), jnp.float32),
            pltpu.VMEM_SHARED((n, DP), jnp.float32),
            pltpu.SemaphoreType.DMA,
            pltpu.SemaphoreType.DMA,
            pltpu.SemaphoreType.DMA,
            pltpu.SemaphoreType.DMA,
            pltpu.SemaphoreType.DMA,
            pltpu.SemaphoreType.DMA,
        ],
        **_SC_PARAMS,
    )


def _tc1_body(deg2_ref, x8_ref, dinv_ref, xs_ref):
    deg = deg2_ref[0] + deg2_ref[1] + 1.0
    dinv = lax.rsqrt(deg)
    dinv_ref[...] = dinv
    xs_ref[...] = dinv * x8_ref[...]


def _tc2_body(dinv_ref, s1_ref, x8_ref, b_ref, bp_ref, h1_ref, xs2_ref):
    dinv = dinv_ref[...]
    agg = dinv * (s1_ref[0] + s1_ref[1]) + dinv * dinv * x8_ref[...]
    h = jnp.dot(agg, b_ref[...], preferred_element_type=jnp.float32) + bp_ref[...]
    h1 = jnp.maximum(h, 0.0)
    h1_ref[...] = h1
    xs2_ref[...] = dinv * h1


def _tc3_body(dinv_ref, s2_ref, h1_ref, b_ref, bp_ref, out_ref):
    dinv = dinv_ref[...]
    agg = dinv * (s2_ref[0] + s2_ref[1]) + dinv * dinv * h1_ref[...]
    z = jnp.dot(agg, b_ref[...], preferred_element_type=jnp.float32) + bp_ref[...]
    lane = lax.broadcasted_iota(jnp.int32, z.shape, 1)
    valid = (lane % 8) < 4
    zm = jnp.where(valid, z, -3e38)
    t = jnp.maximum(zm, pltpu.roll(zm, 127, 1))
    t = jnp.maximum(t, pltpu.roll(t, 126, 1))
    rows_i = lax.broadcasted_iota(jnp.int32, (128, 128), 0)
    cols_i = lax.broadcasted_iota(jnp.int32, (128, 128), 1)
    grp = rows_i // 8 == cols_i // 8
    pb = jnp.where(grp & (rows_i % 8 == 0), 1.0, 0.0)
    msel = jnp.where((lane % 8) == 0, t, 0.0)
    m = jnp.dot(msel, pb, preferred_element_type=jnp.float32)
    ez = jnp.where(valid, jnp.exp(z - m), 0.0)
    ps = jnp.where(grp, 1.0, 0.0)
    ssum = jnp.dot(ez, ps, preferred_element_type=jnp.float32)
    out_ref[...] = z - m - jnp.log(ssum)


def kernel(x, edge_index, W1, b1, W2, b2):
    n, d = x.shape
    e = edge_index.shape[1]
    nv = n * DP // 128
    f32 = jnp.float32
    zeros8 = jnp.zeros((n, DP), f32)
    ones8 = jnp.ones((CHUNK, DP), f32)
    x8 = jnp.concatenate([x, jnp.zeros((n, d), f32)], axis=1)
    x8v = x8.reshape(nv, 128)

    deg2 = _make_degree(n, e)(edge_index, zeros8, ones8)
    deg2v = deg2.reshape(NC, nv, 128)

    dinv8v, xs1v = pl.pallas_call(
        _tc1_body,
        out_shape=[jax.ShapeDtypeStruct((nv, 128), f32)] * 2,
    )(deg2v, x8v)

    s1 = _make_agg(n, e)(edge_index, xs1v.reshape(n, DP), zeros8)

    w1b = jnp.kron(jnp.eye(16, dtype=f32), jnp.pad(W1, ((0, 4), (0, 4))))
    b1p = jnp.tile(jnp.concatenate([b1, jnp.zeros((4,), f32)]), 16).reshape(1, 128)
    h1v, xs2v = pl.pallas_call(
        _tc2_body,
        out_shape=[jax.ShapeDtypeStruct((nv, 128), f32)] * 2,
    )(dinv8v, s1.reshape(NC, nv, 128), x8v, w1b, b1p)

    s2 = _make_agg(n, e)(edge_index, xs2v.reshape(n, DP), zeros8)

    w2b = jnp.kron(jnp.eye(16, dtype=f32), jnp.pad(W2, ((0, 4), (0, 4))))
    b2p = jnp.tile(jnp.concatenate([b2, jnp.zeros((4,), f32)]), 16).reshape(1, 128)
    outv = pl.pallas_call(
        _tc3_body,
        out_shape=jax.ShapeDtypeStruct((nv, 128), f32),
    )(dinv8v, s2.reshape(NC, nv, 128), h1v, w2b, b2p)
    return outv.reshape(n, DP)[:, :4]
